# baseline (speedup 1.0000x reference)
import functools
import math

import jax
import jax.numpy as jnp
from jax import lax
from jax.experimental import pallas as pl
from jax.experimental.pallas import tpu as pltpu
from jax.experimental.pallas import tpu_sc as plsc

N_REAL = 10000
E_REAL = 320000
N_PAD = 10240
NW = 32
EB = 32
E_PAD = 321536
EPW = E_PAD // NW
NBLK = EPW // EB
ROWS_PER_SUB = N_PAD // 16
ACC_W = 144

_mesh = plsc.VectorSubcoreMesh(core_axis_name="c", subcore_axis_name="s",
                               num_cores=2, num_subcores=16)


def _lane_sum(v):
    c = plsc.cumsum(v)
    r = lax.rev(c, (0,))
    lane = lax.broadcasted_iota(jnp.int32, (16,), 0)
    return plsc.cumsum(jnp.where(lane == 0, r, 0.0))


def _attn_compute(split, qd_v, kvs_v, vs_v):
    @plsc.parallel_loop(0, EB, 1, unroll=8)
    def edge_body(e):
        if split:
            accm = qd_v[e, pl.ds(0, 16)] * kvs_v[e, pl.ds(0, 16)]
            for j in range(1, 4):
                accm += qd_v[e, pl.ds(16 * j, 16)] * kvs_v[e, pl.ds(16 * j, 16)]
            accs = qd_v[e, pl.ds(64, 16)] * kvs_v[e, pl.ds(64, 16)]
            for j in range(5, 8):
                accs += qd_v[e, pl.ds(16 * j, 16)] * kvs_v[e, pl.ds(16 * j, 16)]
            exm = jnp.exp(_lane_sum(accm))
            exs = jnp.exp(_lane_sum(accs))
            for j in range(4):
                vs_v[e, pl.ds(16 * j, 16)] = (
                    kvs_v[e, pl.ds(128 + 16 * j, 16)] * exm)
            for j in range(4, 8):
                vs_v[e, pl.ds(16 * j, 16)] = (
                    kvs_v[e, pl.ds(128 + 16 * j, 16)] * exs)
            lane = lax.broadcasted_iota(jnp.int32, (16,), 0)
            sel = jnp.where(lane == 0, exm, jnp.where(lane == 1, exs, 0.0))
            vs_v[e, pl.ds(128, 16)] = kvs_v[e, pl.ds(256, 16)] * sel
        else:
            acc = qd_v[e, pl.ds(0, 16)] * kvs_v[e, pl.ds(0, 16)]
            for j in range(1, 8):
                acc += qd_v[e, pl.ds(16 * j, 16)] * kvs_v[e, pl.ds(16 * j, 16)]
            ex = jnp.exp(_lane_sum(acc))
            for j in range(9):
                vs_v[e, pl.ds(16 * j, 16)] = (
                    kvs_v[e, pl.ds(128 + 16 * j, 16)] * ex)


def _edge_attention_body(split, q_hbm, kv_hbm, eidx_hbm, zeros_hbm,
                         acc_out, s_shared,
                         idx01, dstc0, dstc1, qd0, qd1, kv0, kv1,
                         vs0, vs1, semi0, semg0, semg1,
                         semsc0, semsc1):
    cid = lax.axis_index("c")
    sid = lax.axis_index("s")
    wid = sid * 2 + cid
    row0 = sid * ROWS_PER_SUB
    pltpu.sync_copy(zeros_hbm.at[pl.ds(row0, ROWS_PER_SUB)],
                    s_shared.at[pl.ds(row0, ROWS_PER_SUB)])
    plsc.subcore_barrier()

    def drain_scatters():
        pltpu.make_async_copy(zeros_hbm.at[pl.ds(0, EB)], vs0, semsc0).wait()
        pltpu.make_async_copy(zeros_hbm.at[pl.ds(0, EB)], vs1, semsc1).wait()

    def pair_body(g, carry):
        base0 = wid * EPW + g * (2 * EB)
        di = pltpu.async_copy(eidx_hbm.at[:, pl.ds(base0, 2 * EB)], idx01,
                              semi0)
        pl.when(g > 0)(drain_scatters)
        di.wait()
        g0q = pltpu.async_copy(q_hbm.at[idx01.at[1, pl.ds(0, EB)]], qd0, semg0)
        g0kv = pltpu.async_copy(kv_hbm.at[idx01.at[0, pl.ds(0, EB)]], kv0,
                                semg0)
        g1q = pltpu.async_copy(q_hbm.at[idx01.at[1, pl.ds(EB, EB)]], qd1,
                               semg1)
        g1kv = pltpu.async_copy(kv_hbm.at[idx01.at[0, pl.ds(EB, EB)]], kv1,
                                semg1)
        for j in range(EB // 16):
            dstc0[pl.ds(16 * j, 16)] = idx01[1, pl.ds(16 * j, 16)]
            dstc1[pl.ds(16 * j, 16)] = idx01[1, pl.ds(EB + 16 * j, 16)]
        g0q.wait()
        g0kv.wait()
        _attn_compute(split, qd0, kv0, vs0)
        pltpu.async_copy(vs0, s_shared.at[dstc0], semsc0, add=True)
        g1q.wait()
        g1kv.wait()
        _attn_compute(split, qd1, kv1, vs1)
        pltpu.async_copy(vs1, s_shared.at[dstc1], semsc1, add=True)
        return carry

    lax.fori_loop(0, NBLK // 2, pair_body, 0)
    drain_scatters()
    plsc.subcore_barrier()
    pltpu.sync_copy(s_shared.at[pl.ds(row0, ROWS_PER_SUB)],
                    acc_out.at[cid, pl.ds(row0, ROWS_PER_SUB)])


def _make_edge_attention(split):
    return pl.kernel(
        functools.partial(_edge_attention_body, split),
        out_type=jax.ShapeDtypeStruct((2, N_PAD, ACC_W), jnp.float32),
        mesh=_mesh,
        scratch_types=[
            pltpu.VMEM_SHARED((N_PAD, ACC_W), jnp.float32),
            pltpu.VMEM((2, 2 * EB), jnp.int32),
            pltpu.VMEM((EB,), jnp.int32),
            pltpu.VMEM((EB,), jnp.int32),
            pltpu.VMEM((EB, 128), jnp.float32),
            pltpu.VMEM((EB, 128), jnp.float32),
            pltpu.VMEM((EB, 272), jnp.float32),
            pltpu.VMEM((EB, 272), jnp.float32),
            pltpu.VMEM((EB, ACC_W), jnp.float32),
            pltpu.VMEM((EB, ACC_W), jnp.float32),
            pltpu.SemaphoreType.DMA,
            pltpu.SemaphoreType.DMA,
            pltpu.SemaphoreType.DMA,
            pltpu.SemaphoreType.DMA,
            pltpu.SemaphoreType.DMA,
        ],
        compiler_params=pltpu.CompilerParams(needs_layout_passes=False, use_tc_tiling_on_sc=False),
    )


_edge_attention = _make_edge_attention(False)
_edge_attention_pair = _make_edge_attention(True)


def _recon_body(z_hbm, eidx_hbm, out_hbm,
                idx0, idx1, zd0, zd1, zs0, zs1, lb0, lb1,
                semi0, semi1, semg0, semg1, semo0, semo1):
    cid = lax.axis_index("c")
    sid = lax.axis_index("s")
    wid = sid * 2 + cid

    def dot_block(zd_v, zs_v, lbuf_v):
        @plsc.parallel_loop(0, EB, 1, unroll=8)
        def edge_body(e):
            acc = zd_v[e, pl.ds(0, 16)] * zs_v[e, pl.ds(0, 16)]
            for j in range(1, 4):
                acc += zd_v[e, pl.ds(16 * j, 16)] * zs_v[e, pl.ds(16 * j, 16)]
            lbuf_v[e, pl.ds(0, 16)] = acc

    def pair_body(g, carry):
        base0 = wid * EPW + g * (2 * EB)
        base1 = base0 + EB
        di0 = pltpu.async_copy(eidx_hbm.at[:, pl.ds(base0, EB)], idx0, semi0)
        di1 = pltpu.async_copy(eidx_hbm.at[:, pl.ds(base1, EB)], idx1, semi1)
        di0.wait()
        g0d = pltpu.async_copy(z_hbm.at[idx0.at[1]], zd0, semg0)
        g0s = pltpu.async_copy(z_hbm.at[idx0.at[0]], zs0, semg0)
        di1.wait()
        g1d = pltpu.async_copy(z_hbm.at[idx1.at[1]], zd1, semg1)
        g1s = pltpu.async_copy(z_hbm.at[idx1.at[0]], zs1, semg1)
        g0d.wait()
        g0s.wait()
        dot_block(zd0, zs0, lb0)
        o0 = pltpu.async_copy(lb0, out_hbm.at[pl.ds(base0, EB)], semo0)
        g1d.wait()
        g1s.wait()
        dot_block(zd1, zs1, lb1)
        o1 = pltpu.async_copy(lb1, out_hbm.at[pl.ds(base1, EB)], semo1)
        o0.wait()
        o1.wait()
        return carry

    lax.fori_loop(0, NBLK // 2, pair_body, 0)


_recon_kernel = pl.kernel(
    _recon_body,
    out_type=jax.ShapeDtypeStruct((E_PAD, 16), jnp.float32),
    mesh=_mesh,
    scratch_types=[
        pltpu.VMEM((2, EB), jnp.int32),
        pltpu.VMEM((2, EB), jnp.int32),
        pltpu.VMEM((EB, 64), jnp.float32),
        pltpu.VMEM((EB, 64), jnp.float32),
        pltpu.VMEM((EB, 64), jnp.float32),
        pltpu.VMEM((EB, 64), jnp.float32),
        pltpu.VMEM((EB, 16), jnp.float32),
        pltpu.VMEM((EB, 16), jnp.float32),
        pltpu.SemaphoreType.DMA,
        pltpu.SemaphoreType.DMA,
        pltpu.SemaphoreType.DMA,
        pltpu.SemaphoreType.DMA,
        pltpu.SemaphoreType.DMA,
        pltpu.SemaphoreType.DMA,
    ],
    compiler_params=pltpu.CompilerParams(needs_layout_passes=False, use_tc_tiling_on_sc=False),
)


def _recon_reduce_body(in_ref, out_ref):
    out_ref[...] = jnp.sum(in_ref[...], axis=1, keepdims=True)


_RB = 1024
_GRID = N_PAD // _RB


def _softplus(x):
    return jnp.maximum(x, 0.0) + jnp.log1p(jnp.exp(-jnp.abs(x)))


def _row_spec(w):
    return pl.BlockSpec((_RB, w), lambda i: (i, 0))


def _full_spec(a, b):
    return pl.BlockSpec((a, b), lambda i: (0, 0))


def _acc_spec():
    return pl.BlockSpec((2, _RB, ACC_W), lambda i: (0, i, 0))


def _pad16(k, n_ones):
    lane = lax.broadcasted_iota(jnp.int32, (k.shape[0], 16), 1)
    return (lane < n_ones).astype(jnp.float32)


def _tc1_body(x_ref, h_ref, wphi_ref, bphi_ref, wenc_ref, benc_ref,
              wpr_ref, bpr_ref, wprm_ref, bprm_ref, wprs_ref, bprs_ref,
              phi_ref, q_ref, kv_ref, s_ref, pm_ref, ps_ref):
    x = x_ref[...]
    h = h_ref[...]
    phi = jax.nn.relu(jnp.dot(x, wphi_ref[...],
                              preferred_element_type=jnp.float32) + bphi_ref[...])
    qkvs = (jnp.dot(phi, wenc_ref[:128], preferred_element_type=jnp.float32)
            + jnp.dot(h, wenc_ref[128:], preferred_element_type=jnp.float32)
            + benc_ref[...])
    k = qkvs[:, 128:256]
    v = qkvs[:, 256:384]
    phi_ref[...] = phi
    q_ref[...] = qkvs[:, :128]
    kv_ref[...] = jnp.concatenate([k, v, _pad16(v, 1)], axis=-1)
    s_ref[...] = qkvs[:, 384:]
    prh = jax.nn.relu(jnp.dot(h, wpr_ref[...],
                              preferred_element_type=jnp.float32) + bpr_ref[...])
    pm_ref[...] = jnp.dot(prh, wprm_ref[...],
                          preferred_element_type=jnp.float32) + bprm_ref[...]
    ps_ref[...] = _softplus(jnp.dot(prh, wprs_ref[...],
                                    preferred_element_type=jnp.float32) + bprs_ref[...])


def _tc2_body(acc_ref, se_ref, w2_ref, b2_ref, q2_ref, kv2_ref, s2_ref):
    acc = acc_ref[0] + acc_ref[1]
    agg = acc[:, :128] / (acc[:, 128:129] + 1e-16)
    enc_h = jax.nn.relu(agg + se_ref[...])
    qkvs = jnp.dot(enc_h, w2_ref[...],
                   preferred_element_type=jnp.float32) + b2_ref[...]
    k2 = qkvs[:, 128:256]
    v2 = qkvs[:, 256:384]
    q2_ref[...] = qkvs[:, :128]
    kv2_ref[...] = jnp.concatenate([k2, v2, _pad16(v2, 2)], axis=-1)
    s2_ref[...] = qkvs[:, 384:]


def _tc3_body(acc_ref, s2_ref, phi_ref, h_ref,
              wpz_ref, bpz_ref, wgx_ref, bgx_ref, wgh_ref, bgh_ref,
              em_ref, es_ref,
              qxz_ref, kvxz_ref, sxz_ref, qxr_ref, kvxr_ref, sxr_ref,
              qxh_ref, kvxh_ref, sxh_ref, qhz_ref, kvhz_ref, shz_ref,
              qhr_ref, kvhr_ref, shr_ref):
    acc = acc_ref[0] + acc_ref[1]
    em = acc[:, :64] / (acc[:, 128:129] + 1e-16) + s2_ref[:, :64]
    es = _softplus(acc[:, 64:128] / (acc[:, 129:130] + 1e-16) + s2_ref[:, 64:])
    em_ref[...] = em
    es_ref[...] = es
    phi_z = jax.nn.relu(jnp.dot(em, wpz_ref[...],
                                preferred_element_type=jnp.float32) + bpz_ref[...])
    gx = (jnp.dot(phi_ref[...], wgx_ref[:128], preferred_element_type=jnp.float32)
          + jnp.dot(phi_z, wgx_ref[128:], preferred_element_type=jnp.float32)
          + bgx_ref[...])
    gh = jnp.dot(h_ref[...], wgh_ref[...],
                 preferred_element_type=jnp.float32) + bgh_ref[...]
    for i, (qr, kvr, sr) in enumerate([(qxz_ref, kvxz_ref, sxz_ref),
                                       (qxr_ref, kvxr_ref, sxr_ref),
                                       (qxh_ref, kvxh_ref, sxh_ref)]):
        blk = gx[:, 512 * i:512 * (i + 1)]
        qr[...] = blk[:, :128]
        kvr[...] = jnp.concatenate([blk[:, 128:256], blk[:, 256:384],
                                    _pad16(blk, 1)], axis=-1)
        sr[...] = blk[:, 384:]
    for i, (qr, kvr, sr) in enumerate([(qhz_ref, kvhz_ref, shz_ref),
                                       (qhr_ref, kvhr_ref, shr_ref)]):
        blk = gh[:, 512 * i:512 * (i + 1)]
        qr[...] = blk[:, :128]
        kvr[...] = jnp.concatenate([blk[:, 128:256], blk[:, 256:384],
                                    _pad16(blk, 1)], axis=-1)
        sr[...] = blk[:, 384:]


def _agg(acc, s):
    a = acc[0] + acc[1]
    return a[:, :128] / (a[:, 128:129] + 1e-16) + s


def _tc4_body(axz_ref, ahz_ref, axr_ref, ahr_ref,
              sxz_ref, shz_ref, sxr_ref, shr_ref, h_ref,
              whh_ref, bhh_ref,
              zg_ref, qhh_ref, kvhh_ref, shh_ref):
    z_g = jax.nn.sigmoid(_agg(axz_ref[...], sxz_ref[...])
                         + _agg(ahz_ref[...], shz_ref[...]))
    r_g = jax.nn.sigmoid(_agg(axr_ref[...], sxr_ref[...])
                         + _agg(ahr_ref[...], shr_ref[...]))
    rh = r_g * h_ref[...]
    qkvs = jnp.dot(rh, whh_ref[...],
                   preferred_element_type=jnp.float32) + bhh_ref[...]
    zg_ref[...] = z_g
    qhh_ref[...] = qkvs[:, :128]
    kvhh_ref[...] = jnp.concatenate([qkvs[:, 128:256], qkvs[:, 256:384],
                                     _pad16(qkvs, 1)], axis=-1)
    shh_ref[...] = qkvs[:, 384:]


def _tc5_body(axh_ref, ahh_ref, sxh_ref, shh_ref, zg_ref, h_ref, out_ref):
    h_tilde = jnp.tanh(_agg(axh_ref[...], sxh_ref[...])
                       + _agg(ahh_ref[...], shh_ref[...]))
    z_g = zg_ref[...]
    out_ref[...] = z_g * h_ref[...] + (1.0 - z_g) * h_tilde


def _f32(shape):
    return jax.ShapeDtypeStruct(shape, jnp.float32)


def _qkvs_w(p, scale_q):
    return jnp.concatenate([p["q"]["W"] * scale_q, p["k"]["W"],
                            p["v"]["W"], p["s"]["W"]], axis=1)


def _qkvs_b(p, scale_q):
    return jnp.concatenate([p["q"]["b"] * scale_q, p["k"]["b"],
                            p["v"]["b"], p["s"]["b"]])[None, :]


def kernel(x, h, params, edge_index):
    pr = params
    xp = jnp.pad(x, ((0, N_PAD - N_REAL), (0, 0)))
    hp = jnp.pad(h, ((0, N_PAD - N_REAL), (0, 0)))
    eidx = jnp.pad(edge_index, ((0, 0), (0, E_PAD - E_REAL)),
                   constant_values=N_REAL)
    zeros_acc = jnp.zeros((N_PAD, ACC_W), jnp.float32)

    s128 = 1.0 / math.sqrt(128.0)
    s64 = 1.0 / math.sqrt(64.0)
    w_enc = _qkvs_w(pr["enc"], s128)
    b_enc = _qkvs_b(pr["enc"], s128)
    w2 = jnp.concatenate(
        [pr["enc_mean"]["q"]["W"] * s64, pr["enc_std"]["q"]["W"] * s64,
         pr["enc_mean"]["k"]["W"], pr["enc_std"]["k"]["W"],
         pr["enc_mean"]["v"]["W"], pr["enc_std"]["v"]["W"],
         pr["enc_mean"]["s"]["W"], pr["enc_std"]["s"]["W"]], axis=1)
    b2 = jnp.concatenate(
        [pr["enc_mean"]["q"]["b"] * s64, pr["enc_std"]["q"]["b"] * s64,
         pr["enc_mean"]["k"]["b"], pr["enc_std"]["k"]["b"],
         pr["enc_mean"]["v"]["b"], pr["enc_std"]["v"]["b"],
         pr["enc_mean"]["s"]["b"], pr["enc_std"]["s"]["b"]])[None, :]
    w_gx = jnp.concatenate([_qkvs_w(pr["gru_xz"], s128),
                            _qkvs_w(pr["gru_xr"], s128),
                            _qkvs_w(pr["gru_xh"], s128)], axis=1)
    b_gx = jnp.concatenate([_qkvs_b(pr["gru_xz"], s128),
                            _qkvs_b(pr["gru_xr"], s128),
                            _qkvs_b(pr["gru_xh"], s128)], axis=1)
    w_gh = jnp.concatenate([_qkvs_w(pr["gru_hz"], s128),
                            _qkvs_w(pr["gru_hr"], s128)], axis=1)
    b_gh = jnp.concatenate([_qkvs_b(pr["gru_hz"], s128),
                            _qkvs_b(pr["gru_hr"], s128)], axis=1)
    w_hh = _qkvs_w(pr["gru_hh"], s128)
    b_hh = _qkvs_b(pr["gru_hh"], s128)

    phi_x, q_e, kv_e, s_e, prior_mean, prior_std = pl.pallas_call(
        _tc1_body,
        grid=(_GRID,),
        in_specs=[_row_spec(128), _row_spec(128),
                  _full_spec(128, 128), _full_spec(1, 128),
                  _full_spec(256, 512), _full_spec(1, 512),
                  _full_spec(128, 128), _full_spec(1, 128),
                  _full_spec(128, 64), _full_spec(1, 64),
                  _full_spec(128, 64), _full_spec(1, 64)],
        out_specs=[_row_spec(128), _row_spec(128), _row_spec(272),
                   _row_spec(128), _row_spec(64), _row_spec(64)],
        out_shape=[_f32((N_PAD, 128)), _f32((N_PAD, 128)), _f32((N_PAD, 272)),
                   _f32((N_PAD, 128)), _f32((N_PAD, 64)), _f32((N_PAD, 64))],
    )(xp, hp, pr["phi_x"]["W"], pr["phi_x"]["b"][None, :], w_enc, b_enc,
      pr["prior"]["W"], pr["prior"]["b"][None, :],
      pr["prior_mean"]["W"], pr["prior_mean"]["b"][None, :],
      pr["prior_std"]["W"], pr["prior_std"]["b"][None, :])

    acc_enc = _edge_attention(q_e, kv_e, eidx, zeros_acc)

    q2, kv2, s2 = pl.pallas_call(
        _tc2_body,
        grid=(_GRID,),
        in_specs=[_acc_spec(), _row_spec(128),
                  _full_spec(128, 512), _full_spec(1, 512)],
        out_specs=[_row_spec(128), _row_spec(272), _row_spec(128)],
        out_shape=[_f32((N_PAD, 128)), _f32((N_PAD, 272)), _f32((N_PAD, 128))],
    )(acc_enc, s_e, w2, b2)

    acc_pair = _edge_attention_pair(q2, kv2, eidx, zeros_acc)

    outs3 = pl.pallas_call(
        _tc3_body,
        grid=(_GRID,),
        in_specs=[_acc_spec(), _row_spec(128), _row_spec(128), _row_spec(128),
                  _full_spec(64, 128), _full_spec(1, 128),
                  _full_spec(256, 1536), _full_spec(1, 1536),
                  _full_spec(128, 1024), _full_spec(1, 1024)],
        out_specs=[_row_spec(64), _row_spec(64)]
        + [_row_spec(128), _row_spec(272), _row_spec(128)] * 5,
        out_shape=[_f32((N_PAD, 64)), _f32((N_PAD, 64))]
        + [_f32((N_PAD, 128)), _f32((N_PAD, 272)), _f32((N_PAD, 128))] * 5,
    )(acc_pair, s2, phi_x, hp, pr["phi_z"]["W"], pr["phi_z"]["b"][None, :],
      w_gx, b_gx, w_gh, b_gh)
    (enc_mean, enc_std,
     q_xz, kv_xz, s_xz, q_xr, kv_xr, s_xr, q_xh, kv_xh, s_xh,
     q_hz, kv_hz, s_hz, q_hr, kv_hr, s_hr) = outs3

    recon_part = _recon_kernel(enc_mean, eidx)
    recon2d = pl.pallas_call(
        _recon_reduce_body,
        grid=(157,),
        in_specs=[pl.BlockSpec((2048, 16), lambda i: (i, 0))],
        out_specs=pl.BlockSpec((2048, 1), lambda i: (i, 0)),
        out_shape=_f32((E_PAD, 1)),
    )(recon_part)
    recon = recon2d.reshape(E_PAD)

    acc_xz = _edge_attention(q_xz, kv_xz, eidx, zeros_acc)
    acc_hz = _edge_attention(q_hz, kv_hz, eidx, zeros_acc)
    acc_xr = _edge_attention(q_xr, kv_xr, eidx, zeros_acc)
    acc_hr = _edge_attention(q_hr, kv_hr, eidx, zeros_acc)
    acc_xh = _edge_attention(q_xh, kv_xh, eidx, zeros_acc)

    z_g, q_hh, kv_hh, s_hh = pl.pallas_call(
        _tc4_body,
        grid=(_GRID,),
        in_specs=[_acc_spec()] * 4
        + [_row_spec(128)] * 5
        + [_full_spec(128, 512), _full_spec(1, 512)],
        out_specs=[_row_spec(128), _row_spec(128), _row_spec(272),
                   _row_spec(128)],
        out_shape=[_f32((N_PAD, 128)), _f32((N_PAD, 128)), _f32((N_PAD, 272)),
                   _f32((N_PAD, 128))],
    )(acc_xz, acc_hz, acc_xr, acc_hr, s_xz, s_hz, s_xr, s_hr, hp, w_hh, b_hh)

    acc_hh = _edge_attention(q_hh, kv_hh, eidx, zeros_acc)

    h_out = pl.pallas_call(
        _tc5_body,
        grid=(_GRID,),
        in_specs=[_acc_spec()] * 2 + [_row_spec(128)] * 4,
        out_specs=_row_spec(128),
        out_shape=_f32((N_PAD, 128)),
    )(acc_xh, acc_hh, s_xh, s_hh, z_g, hp)

    return (recon[:E_REAL], h_out[:N_REAL], enc_mean[:N_REAL],
            enc_std[:N_REAL], prior_mean[:N_REAL], prior_std[:N_REAL])


# speedup vs baseline: 1.0298x; 1.0298x over previous
import functools
import math

import jax
import jax.numpy as jnp
from jax import lax
from jax.experimental import pallas as pl
from jax.experimental.pallas import tpu as pltpu
from jax.experimental.pallas import tpu_sc as plsc

N_REAL = 10000
E_REAL = 320000
N_PAD = 10240
NW = 32
EB = 32
E_PAD = 321536
EPW = E_PAD // NW
NBLK = EPW // EB
ROWS_PER_SUB = N_PAD // 16
ACC_W = 144

_mesh = plsc.VectorSubcoreMesh(core_axis_name="c", subcore_axis_name="s",
                               num_cores=2, num_subcores=16)


def _lane_sum(v):
    c = plsc.cumsum(v)
    r = lax.rev(c, (0,))
    lane = lax.broadcasted_iota(jnp.int32, (16,), 0)
    return plsc.cumsum(jnp.where(lane == 0, r, 0.0))


def _attn_compute(split, qd_v, kvs_v, vs_v):
    @plsc.parallel_loop(0, EB, 1, unroll=4)
    def edge_body(e):
        if split:
            accm = qd_v[e, pl.ds(0, 16)] * kvs_v[e, pl.ds(0, 16)]
            for j in range(1, 4):
                accm += qd_v[e, pl.ds(16 * j, 16)] * kvs_v[e, pl.ds(16 * j, 16)]
            accs = qd_v[e, pl.ds(64, 16)] * kvs_v[e, pl.ds(64, 16)]
            for j in range(5, 8):
                accs += qd_v[e, pl.ds(16 * j, 16)] * kvs_v[e, pl.ds(16 * j, 16)]
            exm = jnp.exp(_lane_sum(accm))
            exs = jnp.exp(_lane_sum(accs))
            for j in range(4):
                vs_v[e, pl.ds(16 * j, 16)] = (
                    kvs_v[e, pl.ds(128 + 16 * j, 16)] * exm)
            for j in range(4, 8):
                vs_v[e, pl.ds(16 * j, 16)] = (
                    kvs_v[e, pl.ds(128 + 16 * j, 16)] * exs)
            lane = lax.broadcasted_iota(jnp.int32, (16,), 0)
            sel = jnp.where(lane == 0, exm, jnp.where(lane == 1, exs, 0.0))
            vs_v[e, pl.ds(128, 16)] = kvs_v[e, pl.ds(256, 16)] * sel
        else:
            acc = qd_v[e, pl.ds(0, 16)] * kvs_v[e, pl.ds(0, 16)]
            for j in range(1, 8):
                acc += qd_v[e, pl.ds(16 * j, 16)] * kvs_v[e, pl.ds(16 * j, 16)]
            ex = jnp.exp(_lane_sum(acc))
            for j in range(9):
                vs_v[e, pl.ds(16 * j, 16)] = (
                    kvs_v[e, pl.ds(128 + 16 * j, 16)] * ex)


def _edge_attention_body(split, q_hbm, kv_hbm, eidx_hbm, zeros_hbm,
                         acc_out, s_shared,
                         idx01, dstc0, dstc1, qd0, qd1, kv0, kv1,
                         vs0, vs1, semi0, semg0, semg1,
                         semsc0, semsc1):
    cid = lax.axis_index("c")
    sid = lax.axis_index("s")
    wid = sid * 2 + cid
    row0 = sid * ROWS_PER_SUB
    pltpu.sync_copy(zeros_hbm.at[pl.ds(row0, ROWS_PER_SUB)],
                    s_shared.at[pl.ds(row0, ROWS_PER_SUB)])
    plsc.subcore_barrier()

    def drain_scatters():
        pltpu.make_async_copy(zeros_hbm.at[pl.ds(0, EB)], vs0, semsc0).wait()
        pltpu.make_async_copy(zeros_hbm.at[pl.ds(0, EB)], vs1, semsc1).wait()

    def pair_body(g, carry):
        base0 = wid * EPW + g * (2 * EB)
        di = pltpu.async_copy(eidx_hbm.at[:, pl.ds(base0, 2 * EB)], idx01,
                              semi0)
        pl.when(g > 0)(drain_scatters)
        di.wait()
        g0q = pltpu.async_copy(q_hbm.at[idx01.at[1, pl.ds(0, EB)]], qd0, semg0)
        g0kv = pltpu.async_copy(kv_hbm.at[idx01.at[0, pl.ds(0, EB)]], kv0,
                                semg0)
        g1q = pltpu.async_copy(q_hbm.at[idx01.at[1, pl.ds(EB, EB)]], qd1,
                               semg1)
        g1kv = pltpu.async_copy(kv_hbm.at[idx01.at[0, pl.ds(EB, EB)]], kv1,
                                semg1)
        for j in range(EB // 16):
            dstc0[pl.ds(16 * j, 16)] = idx01[1, pl.ds(16 * j, 16)]
            dstc1[pl.ds(16 * j, 16)] = idx01[1, pl.ds(EB + 16 * j, 16)]
        g0q.wait()
        g0kv.wait()
        _attn_compute(split, qd0, kv0, vs0)
        pltpu.async_copy(vs0, s_shared.at[dstc0], semsc0, add=True)
        g1q.wait()
        g1kv.wait()
        _attn_compute(split, qd1, kv1, vs1)
        pltpu.async_copy(vs1, s_shared.at[dstc1], semsc1, add=True)
        return carry

    lax.fori_loop(0, NBLK // 2, pair_body, 0)
    drain_scatters()
    plsc.subcore_barrier()
    pltpu.sync_copy(s_shared.at[pl.ds(row0, ROWS_PER_SUB)],
                    acc_out.at[cid, pl.ds(row0, ROWS_PER_SUB)])


def _make_edge_attention(split):
    return pl.kernel(
        functools.partial(_edge_attention_body, split),
        out_type=jax.ShapeDtypeStruct((2, N_PAD, ACC_W), jnp.float32),
        mesh=_mesh,
        scratch_types=[
            pltpu.VMEM_SHARED((N_PAD, ACC_W), jnp.float32),
            pltpu.VMEM((2, 2 * EB), jnp.int32),
            pltpu.VMEM((EB,), jnp.int32),
            pltpu.VMEM((EB,), jnp.int32),
            pltpu.VMEM((EB, 128), jnp.float32),
            pltpu.VMEM((EB, 128), jnp.float32),
            pltpu.VMEM((EB, 272), jnp.float32),
            pltpu.VMEM((EB, 272), jnp.float32),
            pltpu.VMEM((EB, ACC_W), jnp.float32),
            pltpu.VMEM((EB, ACC_W), jnp.float32),
            pltpu.SemaphoreType.DMA,
            pltpu.SemaphoreType.DMA,
            pltpu.SemaphoreType.DMA,
            pltpu.SemaphoreType.DMA,
            pltpu.SemaphoreType.DMA,
        ],
        compiler_params=pltpu.CompilerParams(needs_layout_passes=False, use_tc_tiling_on_sc=False),
    )


_edge_attention = _make_edge_attention(False)
_edge_attention_pair = _make_edge_attention(True)


def _recon_body(z_hbm, eidx_hbm, out_hbm,
                idx0, idx1, zd0, zd1, zs0, zs1, lb0, lb1,
                semi0, semi1, semg0, semg1, semo0, semo1):
    cid = lax.axis_index("c")
    sid = lax.axis_index("s")
    wid = sid * 2 + cid

    def dot_block(zd_v, zs_v, lbuf_v):
        @plsc.parallel_loop(0, EB, 1, unroll=4)
        def edge_body(e):
            acc = zd_v[e, pl.ds(0, 16)] * zs_v[e, pl.ds(0, 16)]
            for j in range(1, 4):
                acc += zd_v[e, pl.ds(16 * j, 16)] * zs_v[e, pl.ds(16 * j, 16)]
            lbuf_v[e, pl.ds(0, 16)] = acc

    def pair_body(g, carry):
        base0 = wid * EPW + g * (2 * EB)
        base1 = base0 + EB
        di0 = pltpu.async_copy(eidx_hbm.at[:, pl.ds(base0, EB)], idx0, semi0)
        di1 = pltpu.async_copy(eidx_hbm.at[:, pl.ds(base1, EB)], idx1, semi1)
        di0.wait()
        g0d = pltpu.async_copy(z_hbm.at[idx0.at[1]], zd0, semg0)
        g0s = pltpu.async_copy(z_hbm.at[idx0.at[0]], zs0, semg0)
        di1.wait()
        g1d = pltpu.async_copy(z_hbm.at[idx1.at[1]], zd1, semg1)
        g1s = pltpu.async_copy(z_hbm.at[idx1.at[0]], zs1, semg1)
        g0d.wait()
        g0s.wait()
        dot_block(zd0, zs0, lb0)
        o0 = pltpu.async_copy(lb0, out_hbm.at[pl.ds(base0, EB)], semo0)
        g1d.wait()
        g1s.wait()
        dot_block(zd1, zs1, lb1)
        o1 = pltpu.async_copy(lb1, out_hbm.at[pl.ds(base1, EB)], semo1)
        o0.wait()
        o1.wait()
        return carry

    lax.fori_loop(0, NBLK // 2, pair_body, 0)


_recon_kernel = pl.kernel(
    _recon_body,
    out_type=jax.ShapeDtypeStruct((E_PAD, 16), jnp.float32),
    mesh=_mesh,
    scratch_types=[
        pltpu.VMEM((2, EB), jnp.int32),
        pltpu.VMEM((2, EB), jnp.int32),
        pltpu.VMEM((EB, 64), jnp.float32),
        pltpu.VMEM((EB, 64), jnp.float32),
        pltpu.VMEM((EB, 64), jnp.float32),
        pltpu.VMEM((EB, 64), jnp.float32),
        pltpu.VMEM((EB, 16), jnp.float32),
        pltpu.VMEM((EB, 16), jnp.float32),
        pltpu.SemaphoreType.DMA,
        pltpu.SemaphoreType.DMA,
        pltpu.SemaphoreType.DMA,
        pltpu.SemaphoreType.DMA,
        pltpu.SemaphoreType.DMA,
        pltpu.SemaphoreType.DMA,
    ],
    compiler_params=pltpu.CompilerParams(needs_layout_passes=False, use_tc_tiling_on_sc=False),
)


def _recon_reduce_body(in_ref, out_ref):
    out_ref[...] = jnp.sum(in_ref[...], axis=1, keepdims=True)


_RB = 1024
_GRID = N_PAD // _RB


def _softplus(x):
    return jnp.maximum(x, 0.0) + jnp.log1p(jnp.exp(-jnp.abs(x)))


def _row_spec(w):
    return pl.BlockSpec((_RB, w), lambda i: (i, 0))


def _full_spec(a, b):
    return pl.BlockSpec((a, b), lambda i: (0, 0))


def _acc_spec():
    return pl.BlockSpec((2, _RB, ACC_W), lambda i: (0, i, 0))


def _pad16(k, n_ones):
    lane = lax.broadcasted_iota(jnp.int32, (k.shape[0], 16), 1)
    return (lane < n_ones).astype(jnp.float32)


def _tc1_body(x_ref, h_ref, wphi_ref, bphi_ref, wenc_ref, benc_ref,
              wpr_ref, bpr_ref, wprm_ref, bprm_ref, wprs_ref, bprs_ref,
              phi_ref, q_ref, kv_ref, s_ref, pm_ref, ps_ref):
    x = x_ref[...]
    h = h_ref[...]
    phi = jax.nn.relu(jnp.dot(x, wphi_ref[...],
                              preferred_element_type=jnp.float32) + bphi_ref[...])
    qkvs = (jnp.dot(phi, wenc_ref[:128], preferred_element_type=jnp.float32)
            + jnp.dot(h, wenc_ref[128:], preferred_element_type=jnp.float32)
            + benc_ref[...])
    k = qkvs[:, 128:256]
    v = qkvs[:, 256:384]
    phi_ref[...] = phi
    q_ref[...] = qkvs[:, :128]
    kv_ref[...] = jnp.concatenate([k, v, _pad16(v, 1)], axis=-1)
    s_ref[...] = qkvs[:, 384:]
    prh = jax.nn.relu(jnp.dot(h, wpr_ref[...],
                              preferred_element_type=jnp.float32) + bpr_ref[...])
    pm_ref[...] = jnp.dot(prh, wprm_ref[...],
                          preferred_element_type=jnp.float32) + bprm_ref[...]
    ps_ref[...] = _softplus(jnp.dot(prh, wprs_ref[...],
                                    preferred_element_type=jnp.float32) + bprs_ref[...])


def _tc2_body(acc_ref, se_ref, w2_ref, b2_ref, q2_ref, kv2_ref, s2_ref):
    acc = acc_ref[0] + acc_ref[1]
    agg = acc[:, :128] / (acc[:, 128:129] + 1e-16)
    enc_h = jax.nn.relu(agg + se_ref[...])
    qkvs = jnp.dot(enc_h, w2_ref[...],
                   preferred_element_type=jnp.float32) + b2_ref[...]
    k2 = qkvs[:, 128:256]
    v2 = qkvs[:, 256:384]
    q2_ref[...] = qkvs[:, :128]
    kv2_ref[...] = jnp.concatenate([k2, v2, _pad16(v2, 2)], axis=-1)
    s2_ref[...] = qkvs[:, 384:]


def _tc3_body(acc_ref, s2_ref, phi_ref, h_ref,
              wpz_ref, bpz_ref, wgx_ref, bgx_ref, wgh_ref, bgh_ref,
              em_ref, es_ref,
              qxz_ref, kvxz_ref, sxz_ref, qxr_ref, kvxr_ref, sxr_ref,
              qxh_ref, kvxh_ref, sxh_ref, qhz_ref, kvhz_ref, shz_ref,
              qhr_ref, kvhr_ref, shr_ref):
    acc = acc_ref[0] + acc_ref[1]
    em = acc[:, :64] / (acc[:, 128:129] + 1e-16) + s2_ref[:, :64]
    es = _softplus(acc[:, 64:128] / (acc[:, 129:130] + 1e-16) + s2_ref[:, 64:])
    em_ref[...] = em
    es_ref[...] = es
    phi_z = jax.nn.relu(jnp.dot(em, wpz_ref[...],
                                preferred_element_type=jnp.float32) + bpz_ref[...])
    gx = (jnp.dot(phi_ref[...], wgx_ref[:128], preferred_element_type=jnp.float32)
          + jnp.dot(phi_z, wgx_ref[128:], preferred_element_type=jnp.float32)
          + bgx_ref[...])
    gh = jnp.dot(h_ref[...], wgh_ref[...],
                 preferred_element_type=jnp.float32) + bgh_ref[...]
    for i, (qr, kvr, sr) in enumerate([(qxz_ref, kvxz_ref, sxz_ref),
                                       (qxr_ref, kvxr_ref, sxr_ref),
                                       (qxh_ref, kvxh_ref, sxh_ref)]):
        blk = gx[:, 512 * i:512 * (i + 1)]
        qr[...] = blk[:, :128]
        kvr[...] = jnp.concatenate([blk[:, 128:256], blk[:, 256:384],
                                    _pad16(blk, 1)], axis=-1)
        sr[...] = blk[:, 384:]
    for i, (qr, kvr, sr) in enumerate([(qhz_ref, kvhz_ref, shz_ref),
                                       (qhr_ref, kvhr_ref, shr_ref)]):
        blk = gh[:, 512 * i:512 * (i + 1)]
        qr[...] = blk[:, :128]
        kvr[...] = jnp.concatenate([blk[:, 128:256], blk[:, 256:384],
                                    _pad16(blk, 1)], axis=-1)
        sr[...] = blk[:, 384:]


def _agg(acc, s):
    a = acc[0] + acc[1]
    return a[:, :128] / (a[:, 128:129] + 1e-16) + s


def _tc4_body(axz_ref, ahz_ref, axr_ref, ahr_ref,
              sxz_ref, shz_ref, sxr_ref, shr_ref, h_ref,
              whh_ref, bhh_ref,
              zg_ref, qhh_ref, kvhh_ref, shh_ref):
    z_g = jax.nn.sigmoid(_agg(axz_ref[...], sxz_ref[...])
                         + _agg(ahz_ref[...], shz_ref[...]))
    r_g = jax.nn.sigmoid(_agg(axr_ref[...], sxr_ref[...])
                         + _agg(ahr_ref[...], shr_ref[...]))
    rh = r_g * h_ref[...]
    qkvs = jnp.dot(rh, whh_ref[...],
                   preferred_element_type=jnp.float32) + bhh_ref[...]
    zg_ref[...] = z_g
    qhh_ref[...] = qkvs[:, :128]
    kvhh_ref[...] = jnp.concatenate([qkvs[:, 128:256], qkvs[:, 256:384],
                                     _pad16(qkvs, 1)], axis=-1)
    shh_ref[...] = qkvs[:, 384:]


def _tc5_body(axh_ref, ahh_ref, sxh_ref, shh_ref, zg_ref, h_ref, out_ref):
    h_tilde = jnp.tanh(_agg(axh_ref[...], sxh_ref[...])
                       + _agg(ahh_ref[...], shh_ref[...]))
    z_g = zg_ref[...]
    out_ref[...] = z_g * h_ref[...] + (1.0 - z_g) * h_tilde


def _f32(shape):
    return jax.ShapeDtypeStruct(shape, jnp.float32)


def _qkvs_w(p, scale_q):
    return jnp.concatenate([p["q"]["W"] * scale_q, p["k"]["W"],
                            p["v"]["W"], p["s"]["W"]], axis=1)


def _qkvs_b(p, scale_q):
    return jnp.concatenate([p["q"]["b"] * scale_q, p["k"]["b"],
                            p["v"]["b"], p["s"]["b"]])[None, :]


def kernel(x, h, params, edge_index):
    pr = params
    xp = jnp.pad(x, ((0, N_PAD - N_REAL), (0, 0)))
    hp = jnp.pad(h, ((0, N_PAD - N_REAL), (0, 0)))
    eidx = jnp.pad(edge_index, ((0, 0), (0, E_PAD - E_REAL)),
                   constant_values=N_REAL)
    zeros_acc = jnp.zeros((N_PAD, ACC_W), jnp.float32)

    s128 = 1.0 / math.sqrt(128.0)
    s64 = 1.0 / math.sqrt(64.0)
    w_enc = _qkvs_w(pr["enc"], s128)
    b_enc = _qkvs_b(pr["enc"], s128)
    w2 = jnp.concatenate(
        [pr["enc_mean"]["q"]["W"] * s64, pr["enc_std"]["q"]["W"] * s64,
         pr["enc_mean"]["k"]["W"], pr["enc_std"]["k"]["W"],
         pr["enc_mean"]["v"]["W"], pr["enc_std"]["v"]["W"],
         pr["enc_mean"]["s"]["W"], pr["enc_std"]["s"]["W"]], axis=1)
    b2 = jnp.concatenate(
        [pr["enc_mean"]["q"]["b"] * s64, pr["enc_std"]["q"]["b"] * s64,
         pr["enc_mean"]["k"]["b"], pr["enc_std"]["k"]["b"],
         pr["enc_mean"]["v"]["b"], pr["enc_std"]["v"]["b"],
         pr["enc_mean"]["s"]["b"], pr["enc_std"]["s"]["b"]])[None, :]
    w_gx = jnp.concatenate([_qkvs_w(pr["gru_xz"], s128),
                            _qkvs_w(pr["gru_xr"], s128),
                            _qkvs_w(pr["gru_xh"], s128)], axis=1)
    b_gx = jnp.concatenate([_qkvs_b(pr["gru_xz"], s128),
                            _qkvs_b(pr["gru_xr"], s128),
                            _qkvs_b(pr["gru_xh"], s128)], axis=1)
    w_gh = jnp.concatenate([_qkvs_w(pr["gru_hz"], s128),
                            _qkvs_w(pr["gru_hr"], s128)], axis=1)
    b_gh = jnp.concatenate([_qkvs_b(pr["gru_hz"], s128),
                            _qkvs_b(pr["gru_hr"], s128)], axis=1)
    w_hh = _qkvs_w(pr["gru_hh"], s128)
    b_hh = _qkvs_b(pr["gru_hh"], s128)

    phi_x, q_e, kv_e, s_e, prior_mean, prior_std = pl.pallas_call(
        _tc1_body,
        grid=(_GRID,),
        in_specs=[_row_spec(128), _row_spec(128),
                  _full_spec(128, 128), _full_spec(1, 128),
                  _full_spec(256, 512), _full_spec(1, 512),
                  _full_spec(128, 128), _full_spec(1, 128),
                  _full_spec(128, 64), _full_spec(1, 64),
                  _full_spec(128, 64), _full_spec(1, 64)],
        out_specs=[_row_spec(128), _row_spec(128), _row_spec(272),
                   _row_spec(128), _row_spec(64), _row_spec(64)],
        out_shape=[_f32((N_PAD, 128)), _f32((N_PAD, 128)), _f32((N_PAD, 272)),
                   _f32((N_PAD, 128)), _f32((N_PAD, 64)), _f32((N_PAD, 64))],
    )(xp, hp, pr["phi_x"]["W"], pr["phi_x"]["b"][None, :], w_enc, b_enc,
      pr["prior"]["W"], pr["prior"]["b"][None, :],
      pr["prior_mean"]["W"], pr["prior_mean"]["b"][None, :],
      pr["prior_std"]["W"], pr["prior_std"]["b"][None, :])

    acc_enc = _edge_attention(q_e, kv_e, eidx, zeros_acc)

    q2, kv2, s2 = pl.pallas_call(
        _tc2_body,
        grid=(_GRID,),
        in_specs=[_acc_spec(), _row_spec(128),
                  _full_spec(128, 512), _full_spec(1, 512)],
        out_specs=[_row_spec(128), _row_spec(272), _row_spec(128)],
        out_shape=[_f32((N_PAD, 128)), _f32((N_PAD, 272)), _f32((N_PAD, 128))],
    )(acc_enc, s_e, w2, b2)

    acc_pair = _edge_attention_pair(q2, kv2, eidx, zeros_acc)

    outs3 = pl.pallas_call(
        _tc3_body,
        grid=(_GRID,),
        in_specs=[_acc_spec(), _row_spec(128), _row_spec(128), _row_spec(128),
                  _full_spec(64, 128), _full_spec(1, 128),
                  _full_spec(256, 1536), _full_spec(1, 1536),
                  _full_spec(128, 1024), _full_spec(1, 1024)],
        out_specs=[_row_spec(64), _row_spec(64)]
        + [_row_spec(128), _row_spec(272), _row_spec(128)] * 5,
        out_shape=[_f32((N_PAD, 64)), _f32((N_PAD, 64))]
        + [_f32((N_PAD, 128)), _f32((N_PAD, 272)), _f32((N_PAD, 128))] * 5,
    )(acc_pair, s2, phi_x, hp, pr["phi_z"]["W"], pr["phi_z"]["b"][None, :],
      w_gx, b_gx, w_gh, b_gh)
    (enc_mean, enc_std,
     q_xz, kv_xz, s_xz, q_xr, kv_xr, s_xr, q_xh, kv_xh, s_xh,
     q_hz, kv_hz, s_hz, q_hr, kv_hr, s_hr) = outs3

    recon_part = _recon_kernel(enc_mean, eidx)
    recon2d = pl.pallas_call(
        _recon_reduce_body,
        grid=(157,),
        in_specs=[pl.BlockSpec((2048, 16), lambda i: (i, 0))],
        out_specs=pl.BlockSpec((2048, 1), lambda i: (i, 0)),
        out_shape=_f32((E_PAD, 1)),
    )(recon_part)
    recon = recon2d.reshape(E_PAD)

    acc_xz = _edge_attention(q_xz, kv_xz, eidx, zeros_acc)
    acc_hz = _edge_attention(q_hz, kv_hz, eidx, zeros_acc)
    acc_xr = _edge_attention(q_xr, kv_xr, eidx, zeros_acc)
    acc_hr = _edge_attention(q_hr, kv_hr, eidx, zeros_acc)
    acc_xh = _edge_attention(q_xh, kv_xh, eidx, zeros_acc)

    z_g, q_hh, kv_hh, s_hh = pl.pallas_call(
        _tc4_body,
        grid=(_GRID,),
        in_specs=[_acc_spec()] * 4
        + [_row_spec(128)] * 5
        + [_full_spec(128, 512), _full_spec(1, 512)],
        out_specs=[_row_spec(128), _row_spec(128), _row_spec(272),
                   _row_spec(128)],
        out_shape=[_f32((N_PAD, 128)), _f32((N_PAD, 128)), _f32((N_PAD, 272)),
                   _f32((N_PAD, 128))],
    )(acc_xz, acc_hz, acc_xr, acc_hr, s_xz, s_hz, s_xr, s_hr, hp, w_hh, b_hh)

    acc_hh = _edge_attention(q_hh, kv_hh, eidx, zeros_acc)

    h_out = pl.pallas_call(
        _tc5_body,
        grid=(_GRID,),
        in_specs=[_acc_spec()] * 2 + [_row_spec(128)] * 4,
        out_specs=_row_spec(128),
        out_shape=_f32((N_PAD, 128)),
    )(acc_xh, acc_hh, s_xh, s_hh, z_g, hp)

    return (recon[:E_REAL], h_out[:N_REAL], enc_mean[:N_REAL],
            enc_std[:N_REAL], prior_mean[:N_REAL], prior_std[:N_REAL])
